# initial kernel scaffold (unmeasured)
import jax
import jax.numpy as jnp
from jax import lax
from jax.experimental import pallas as pl
from jax.experimental.pallas import tpu as pltpu

N_DEV = 4
B, SQ, SKV, HQ, DH = 2, 128, 512, 4, 64
DMODEL = 512
DQK = HQ * DH
BLK = 64


def kernel(x, Wq, K_ext, V_ext, Wo):
    skv_per = K_ext.shape[1]
    k2 = K_ext.reshape(B, skv_per, DQK)
    v2 = V_ext.reshape(B, skv_per, DQK)

    def body(x_ref, wq_ref, k_ref, v_ref, wo_ref, out_ref,
             comm_ref, send_sems, recv_sems):
        my = lax.axis_index("i")
        left = lax.rem(my + N_DEV - 1, N_DEV)
        right = lax.rem(my + 1, N_DEV)

        barrier_sem = pltpu.get_barrier_semaphore()
        for nbr in (left, right):
            pl.semaphore_signal(barrier_sem, inc=1, device_id=(nbr,),
                                device_id_type=pl.DeviceIdType.MESH)
        pl.semaphore_wait(barrier_sem, 2)

        comm_ref[0, 0] = k_ref[...]
        comm_ref[0, 1] = v_ref[...]

        for h in range(N_DEV - 1):
            rdma = pltpu.make_async_remote_copy(
                src_ref=comm_ref.at[h],
                dst_ref=comm_ref.at[h + 1],
                send_sem=send_sems.at[h],
                recv_sem=recv_sems.at[h],
                device_id=(right,),
                device_id_type=pl.DeviceIdType.MESH,
            )
            rdma.start()
            rdma.wait()

        qb = lax.broadcasted_iota(jnp.int32, (SQ, N_DEV * skv_per), 0) // BLK
        col = lax.broadcasted_iota(jnp.int32, (SQ, N_DEV * skv_per), 1)
        origin = lax.rem(my - col // skv_per + N_DEV, N_DEV)
        kb = 2 * origin + (col % skv_per) // BLK
        mask = (qb == kb) | (lax.rem(kb, 4) == lax.rem(qb, 4))

        for b in range(B):
            q_b = jnp.dot(x_ref[b], wq_ref[...],
                          preferred_element_type=jnp.float32)
            ctx_parts = []
            for hq in range(HQ):
                cs = hq * DH
                q_bh = q_b[:, cs:cs + DH]
                k_bh = jnp.concatenate(
                    [comm_ref[r, 0, b, :, cs:cs + DH] for r in range(N_DEV)],
                    axis=0)
                v_bh = jnp.concatenate(
                    [comm_ref[r, 1, b, :, cs:cs + DH] for r in range(N_DEV)],
                    axis=0)
                s = lax.dot_general(q_bh, k_bh, (((1,), (1,)), ((), ())),
                                    preferred_element_type=jnp.float32) * 0.125
                s = jnp.where(mask, s, -1e9)
                m = jnp.max(s, axis=1, keepdims=True)
                w = jnp.exp(s - m)
                w = w / jnp.sum(w, axis=1, keepdims=True)
                ctx_parts.append(
                    jnp.dot(w, v_bh, preferred_element_type=jnp.float32))
            ctx_b = jnp.concatenate(ctx_parts, axis=1)
            out_ref[b] = jnp.dot(ctx_b, wo_ref[...],
                                 preferred_element_type=jnp.float32)

    return pl.pallas_call(
        body,
        out_shape=jax.ShapeDtypeStruct((B, SQ, DMODEL), jnp.float32),
        in_specs=[pl.BlockSpec(memory_space=pltpu.VMEM)] * 5,
        out_specs=pl.BlockSpec(memory_space=pltpu.VMEM),
        scratch_shapes=[
            pltpu.VMEM((N_DEV, 2, B, skv_per, DQK), jnp.float32),
            pltpu.SemaphoreType.DMA((N_DEV - 1,)),
            pltpu.SemaphoreType.DMA((N_DEV - 1,)),
        ],
        compiler_params=pltpu.CompilerParams(collective_id=0),
    )(x, k2 * 0 + k2 and None or x, Wq, k2, v2, Wo)


# baseline (device time: 33418 ns/iter reference)
import jax
import jax.numpy as jnp
from jax import lax
from jax.experimental import pallas as pl
from jax.experimental.pallas import tpu as pltpu

N_DEV = 4
B, SQ, SKV, HQ, DH = 2, 128, 512, 4, 64
DMODEL = 512
DQK = HQ * DH
BLK = 64


def kernel(x, Wq, K_ext, V_ext, Wo):
    skv_per = K_ext.shape[1]
    k2 = K_ext.reshape(B, skv_per, DQK)
    v2 = V_ext.reshape(B, skv_per, DQK)

    def body(x_ref, wq_ref, k_ref, v_ref, wo_ref, out_ref,
             comm_ref, send_sems, recv_sems):
        my = lax.axis_index("i")
        left = lax.rem(my + N_DEV - 1, N_DEV)
        right = lax.rem(my + 1, N_DEV)

        barrier_sem = pltpu.get_barrier_semaphore()
        for nbr in (left, right):
            pl.semaphore_signal(barrier_sem, inc=1, device_id=(nbr,),
                                device_id_type=pl.DeviceIdType.MESH)
        pl.semaphore_wait(barrier_sem, 2)

        comm_ref[0, 0] = k_ref[...]
        comm_ref[0, 1] = v_ref[...]

        for h in range(N_DEV - 1):
            rdma = pltpu.make_async_remote_copy(
                src_ref=comm_ref.at[h],
                dst_ref=comm_ref.at[h + 1],
                send_sem=send_sems.at[h],
                recv_sem=recv_sems.at[h],
                device_id=(right,),
                device_id_type=pl.DeviceIdType.MESH,
            )
            rdma.start()
            rdma.wait()

        qb = lax.broadcasted_iota(jnp.int32, (SQ, N_DEV * skv_per), 0) // BLK
        col = lax.broadcasted_iota(jnp.int32, (SQ, N_DEV * skv_per), 1)
        origin = lax.rem(my - col // skv_per + N_DEV, N_DEV)
        kb = 2 * origin + (col % skv_per) // BLK
        mask = (qb == kb) | (lax.rem(kb, 4) == lax.rem(qb, 4))

        for b in range(B):
            q_b = jnp.dot(x_ref[b], wq_ref[...],
                          preferred_element_type=jnp.float32)
            ctx_parts = []
            for hq in range(HQ):
                cs = hq * DH
                q_bh = q_b[:, cs:cs + DH]
                k_bh = jnp.concatenate(
                    [comm_ref[r, 0, b, :, cs:cs + DH] for r in range(N_DEV)],
                    axis=0)
                v_bh = jnp.concatenate(
                    [comm_ref[r, 1, b, :, cs:cs + DH] for r in range(N_DEV)],
                    axis=0)
                s = lax.dot_general(q_bh, k_bh, (((1,), (1,)), ((), ())),
                                    preferred_element_type=jnp.float32) * 0.125
                s = jnp.where(mask, s, -1e9)
                m = jnp.max(s, axis=1, keepdims=True)
                w = jnp.exp(s - m)
                w = w / jnp.sum(w, axis=1, keepdims=True)
                ctx_parts.append(
                    jnp.dot(w, v_bh, preferred_element_type=jnp.float32))
            ctx_b = jnp.concatenate(ctx_parts, axis=1)
            out_ref[b] = jnp.dot(ctx_b, wo_ref[...],
                                 preferred_element_type=jnp.float32)

    return pl.pallas_call(
        body,
        out_shape=jax.ShapeDtypeStruct((B, SQ, DMODEL), jnp.float32),
        in_specs=[pl.BlockSpec(memory_space=pltpu.VMEM)] * 5,
        out_specs=pl.BlockSpec(memory_space=pltpu.VMEM),
        scratch_shapes=[
            pltpu.VMEM((N_DEV, 2, B, skv_per, DQK), jnp.float32),
            pltpu.SemaphoreType.DMA((N_DEV - 1,)),
            pltpu.SemaphoreType.DMA((N_DEV - 1,)),
        ],
        compiler_params=pltpu.CompilerParams(collective_id=0),
    )(x, Wq, k2, v2, Wo)


# device time: 19690 ns/iter; 1.6972x vs baseline; 1.6972x over previous
import jax
import jax.numpy as jnp
from jax import lax
from jax.experimental import pallas as pl
from jax.experimental.pallas import tpu as pltpu

N_DEV = 4
B, SQ, SKV, HQ, DH = 2, 128, 512, 4, 64
DMODEL = 512
DQK = HQ * DH
BLK = 64
SRC_DEVS = (0, 2)


def kernel(x, Wq, K_ext, V_ext, Wo):
    skv_per = K_ext.shape[1]
    k2 = K_ext.reshape(B, skv_per, DQK)
    v2 = V_ext.reshape(B, skv_per, DQK)

    def body(x_ref, wq_ref, k_ref, v_ref, wo_ref, out_ref,
             pctx_ref, pstat_ref, send_sems, recv_sems):
        my = lax.axis_index("i")

        barrier_sem = pltpu.get_barrier_semaphore()
        for d in range(1, N_DEV):
            pl.semaphore_signal(barrier_sem, inc=1,
                                device_id=(lax.rem(my + d, N_DEV),),
                                device_id_type=pl.DeviceIdType.MESH)
        pl.semaphore_wait(barrier_sem, N_DEV - 1)

        def partial_and_send(slot, targets):
            diag = (lax.broadcasted_iota(jnp.int32, (SQ, skv_per), 0) // BLK
                    == lax.broadcasted_iota(jnp.int32, (SQ, skv_per), 1) // BLK)
            for b in range(B):
                q_b = jnp.dot(x_ref[b], wq_ref[...],
                              preferred_element_type=jnp.float32)
                uctx, ms, ls = [], [], []
                for hq in range(HQ):
                    cs = hq * DH
                    s = lax.dot_general(
                        q_b[:, cs:cs + DH], k_ref[b, :, cs:cs + DH],
                        (((1,), (1,)), ((), ())),
                        preferred_element_type=jnp.float32) * 0.125
                    s = jnp.where(diag, s, -1e9)
                    m = jnp.max(s, axis=1, keepdims=True)
                    w = jnp.exp(s - m)
                    ls.append(jnp.sum(w, axis=1, keepdims=True))
                    ms.append(m)
                    uctx.append(jnp.dot(w, v_ref[b, :, cs:cs + DH],
                                        preferred_element_type=jnp.float32))
                pctx_ref[slot, b] = jnp.concatenate(uctx, axis=1)
                pstat_ref[slot, b, :, 0:HQ] = jnp.concatenate(ms, axis=1)
                pstat_ref[slot, b, :, HQ:2 * HQ] = jnp.concatenate(ls, axis=1)
            rdmas = []
            for di, tgt in enumerate(targets):
                for t, ref_ in enumerate((pctx_ref, pstat_ref)):
                    r = pltpu.make_async_remote_copy(
                        src_ref=ref_.at[slot], dst_ref=ref_.at[slot],
                        send_sem=send_sems.at[di, t],
                        recv_sem=recv_sems.at[slot, t],
                        device_id=(tgt,),
                        device_id_type=pl.DeviceIdType.MESH)
                    r.start()
                    rdmas.append(r)
            return rdmas

        def wait_recv_slot(slot):
            for t, ref_ in enumerate((pctx_ref, pstat_ref)):
                pltpu.make_async_remote_copy(
                    src_ref=ref_.at[slot], dst_ref=ref_.at[slot],
                    send_sem=send_sems.at[0, t],
                    recv_sem=recv_sems.at[slot, t],
                    device_id=(my,),
                    device_id_type=pl.DeviceIdType.MESH).wait_recv()

        @pl.when(my == 0)
        def _():
            rdmas = partial_and_send(0, targets=(2, 1, 3))
            wait_recv_slot(1)
            for r in rdmas:
                r.wait_send()

        @pl.when(my == 2)
        def _():
            rdmas = partial_and_send(1, targets=(0, 3, 1))
            wait_recv_slot(0)
            for r in rdmas:
                r.wait_send()

        @pl.when((my == 1) | (my == 3))
        def _():
            wait_recv_slot(0)
            wait_recv_slot(1)

        hcol = lax.broadcasted_iota(jnp.int32, (HQ, DQK), 1) // DH
        hrow = lax.broadcasted_iota(jnp.int32, (HQ, DQK), 0)
        expand = jnp.where(hcol == hrow, 1.0, 0.0).astype(jnp.float32)
        for b in range(B):
            m0 = pstat_ref[0, b, :, 0:HQ]
            l0 = pstat_ref[0, b, :, HQ:2 * HQ]
            m2 = pstat_ref[1, b, :, 0:HQ]
            l2 = pstat_ref[1, b, :, HQ:2 * HQ]
            mx = jnp.maximum(m0, m2)
            a0 = jnp.exp(m0 - mx)
            a2 = jnp.exp(m2 - mx)
            li = 1.0 / (l0 * a0 + l2 * a2)
            f0 = jnp.dot(a0 * li, expand,
                         preferred_element_type=jnp.float32)
            f2 = jnp.dot(a2 * li, expand,
                         preferred_element_type=jnp.float32)
            ctx = pctx_ref[0, b] * f0 + pctx_ref[1, b] * f2
            out_ref[b] = jnp.dot(ctx, wo_ref[...],
                                 preferred_element_type=jnp.float32)

    return pl.pallas_call(
        body,
        out_shape=jax.ShapeDtypeStruct((B, SQ, DMODEL), jnp.float32),
        in_specs=[pl.BlockSpec(memory_space=pltpu.VMEM)] * 5,
        out_specs=pl.BlockSpec(memory_space=pltpu.VMEM),
        scratch_shapes=[
            pltpu.VMEM((2, B, SQ, DQK), jnp.float32),
            pltpu.VMEM((2, B, SQ, 2 * HQ), jnp.float32),
            pltpu.SemaphoreType.DMA((3, 2)),
            pltpu.SemaphoreType.DMA((2, 2)),
        ],
        compiler_params=pltpu.CompilerParams(collective_id=0),
    )(x, Wq, k2, v2, Wo)
